# baseline (speedup 1.0000x reference)
import functools

import jax
import jax.numpy as jnp
from jax import lax
from jax.experimental import pallas as pl
from jax.experimental.pallas import tpu as pltpu
from jax.experimental.pallas import tpu_sc as plsc

NPI, NAU, NPUB = 10000, 20000, 50000
DIM, HID, NOUT, NLAYER = 128, 128, 4, 2
NC, NS = 2, 16
FS = 32
KCH = 384
KCNT = 192
BROW = 2000
ZROWS = 3200

NPAD_PI, NPAD_AU, NPAD_PUB = 10112, 20096, 50048

_RELS = (
    ("ap", 40000, 43008, NPI, NPAD_PI, NAU),
    ("pa", 40000, 43008, NAU, NPAD_AU, NPI),
    ("ra", 160000, 165888, NAU, NPAD_AU, NPUB),
    ("aw", 160000, 165888, NPUB, NPAD_PUB, NAU),
    ("cc", 200000, 202752, NPUB, NPAD_PUB, NPUB),
)
_RKEYS = tuple(r[0] for r in _RELS)

_SC_PARAMS = pltpu.CompilerParams(use_tc_tiling_on_sc=False)


@functools.cache
def _mesh():
    return plsc.VectorSubcoreMesh(core_axis_name="c", subcore_axis_name="s")


def _counts_body(*refs):
    it = iter(refs)
    dsts = {k: next(it) for k in _RKEYS}
    ones_hbm = next(it)
    zeros_hbm = next(it)
    outs = {k: next(it) for k in _RKEYS}
    didx, ones_v, cacc, sem = list(it)

    c = lax.axis_index("c")
    s = lax.axis_index("s")
    pltpu.sync_copy(ones_hbm, ones_v)

    for key, e, ep, nd, npad, ns_ in _RELS:
        steps = ep // (NC * NS * KCNT)
        eps = steps * KCNT
        base = (c * NS + s) * eps
        pltpu.sync_copy(dsts[key].at[pl.ds(base, eps)], didx.at[pl.ds(0, eps)])
        rps = npad // NS
        pltpu.sync_copy(zeros_hbm.at[pl.ds(0, rps)], cacc.at[pl.ds(s * rps, rps)])
        plsc.subcore_barrier()

        def step(j, carry):
            pltpu.sync_copy(ones_v, cacc.at[didx.at[pl.ds(j * KCNT, KCNT)]], add=True)
            return carry

        lax.fori_loop(0, steps, step, 0)
        plsc.subcore_barrier()
        pltpu.sync_copy(cacc.at[pl.ds(s * rps, rps)],
                        outs[key].at[c, pl.ds(s * rps, rps)])
        plsc.subcore_barrier()


def _counts_call(dsts, ones8, zeros8):
    out_types = tuple(jax.ShapeDtypeStruct((NC, npad, 8), jnp.float32)
                      for _, _, _, _, npad, _ in _RELS)
    max_eps_c = max(ep // (NC * NS) for _, _, ep, _, _, _ in _RELS)
    kern = functools.partial(
        pl.kernel,
        out_type=out_types,
        mesh=_mesh(),
        scratch_types=[
            pltpu.VMEM((max_eps_c,), jnp.int32),
            pltpu.VMEM((KCNT, 8), jnp.float32),
            pltpu.VMEM_SHARED((NPAD_PUB, 8), jnp.float32),
            pltpu.SemaphoreType.DMA,
        ],
        compiler_params=_SC_PARAMS,
    )(_counts_body)
    outs = kern(*[dsts[k] for k in _RKEYS], ones8, zeros8)
    return dict(zip(_RKEYS, outs))


def _segsum_body(*refs):
    it = iter(refs)
    tables = {k: (next(it), next(it)) for k in _RKEYS}
    edges = {k: (next(it), next(it)) for k in _RKEYS}
    zeros_hbm = next(it)
    outs = {k: (next(it), next(it)) for k in _RKEYS}
    sidxA, didxA, rowsA, sidxB, didxB, rowsB, acc, semA, semB = list(it)

    c = lax.axis_index("c")
    s = lax.axis_index("s")

    for key, e, ep, nd, npad, ns_ in _RELS:
        steps = ep // (NS * KCH)
        src_hbm, dst_hbm = edges[key]
        rps = npad // NS

        def fetch_idx(j, si, di, _src=src_hbm, _dst=dst_hbm, _steps=steps, _ep=ep):
            base = (s * _steps + j) * KCH
            pltpu.sync_copy(_src.at[pl.ds(c * (_ep + 2 * KCH) + base, KCH)], si)
            pltpu.sync_copy(_dst.at[pl.ds(base, KCH)], di)

        for p in range(2):
            tbl = tables[key][p]
            out = outs[key][p]
            pltpu.sync_copy(zeros_hbm.at[pl.ds(0, rps)], acc.at[pl.ds(s * rps, rps)])
            plsc.subcore_barrier()

            fetch_idx(0, sidxA, didxA)
            pltpu.async_copy(tbl.at[sidxA], rowsA, semA)
            fetch_idx(1, sidxB, didxB)

            def step2(jj, carry, _tbl=tbl, _fi=fetch_idx):
                pltpu.make_async_copy(_tbl.at[sidxA], rowsA, semA).wait()
                pltpu.async_copy(_tbl.at[sidxB], rowsB, semB)
                pltpu.sync_copy(rowsA, acc.at[didxA], add=True)
                _fi(2 * jj + 2, sidxA, didxA)
                pltpu.async_copy(_tbl.at[sidxA], rowsA, semA)
                pltpu.make_async_copy(_tbl.at[sidxB], rowsB, semB).wait()
                pltpu.sync_copy(rowsB, acc.at[didxB], add=True)
                _fi(2 * jj + 3, sidxB, didxB)
                return carry

            lax.fori_loop(0, steps // 2, step2, 0)
            pltpu.make_async_copy(tbl.at[sidxA], rowsA, semA).wait()
            if steps % 2 == 1:
                pltpu.sync_copy(rowsA, acc.at[didxA], add=True)
            plsc.subcore_barrier()
            pltpu.sync_copy(acc.at[pl.ds(s * rps, rps)],
                            out.at[c, pl.ds(s * rps, rps)])
            plsc.subcore_barrier()


def _segsum_call(tables, srcs2, dsts, zeros32):
    out_types = []
    for _, _, _, _, npad, _ in _RELS:
        out_types += [jax.ShapeDtypeStruct((NC, npad, FS), jnp.float32)] * 2
    kern = functools.partial(
        pl.kernel,
        out_type=tuple(out_types),
        mesh=_mesh(),
        scratch_types=[
            pltpu.VMEM((KCH,), jnp.int32),
            pltpu.VMEM((KCH,), jnp.int32),
            pltpu.VMEM((KCH, FS), jnp.float32),
            pltpu.VMEM((KCH,), jnp.int32),
            pltpu.VMEM((KCH,), jnp.int32),
            pltpu.VMEM((KCH, FS), jnp.float32),
            pltpu.VMEM_SHARED((NPAD_PUB, FS), jnp.float32),
            pltpu.SemaphoreType.DMA,
            pltpu.SemaphoreType.DMA,
        ],
        compiler_params=_SC_PARAMS,
    )(_segsum_body)
    args = []
    for k in _RKEYS:
        args += list(tables[k])
    for k in _RKEYS:
        args += [srcs2[k], dsts[k]]
    args.append(zeros32)
    flat = kern(*args)
    return {k: (flat[2 * i], flat[2 * i + 1]) for i, k in enumerate(_RKEYS)}


def _row_spec(w):
    return pl.BlockSpec((BROW, w), lambda i: (i, 0))


def _full_spec(h, w):
    return pl.BlockSpec((h, w), lambda i: (0, 0))


_CNT_SPEC = pl.BlockSpec((NC, BROW, 8), lambda i: (0, i, 0))
_AGG_SPEC = pl.BlockSpec((NC, BROW, FS), lambda i: (0, i, 0))


def _emit_table(t_p0, t_p1, y):
    t_p0[0, ...] = y[:, 0 * FS:1 * FS]
    t_p0[1, ...] = y[:, 1 * FS:2 * FS]
    t_p1[0, ...] = y[:, 2 * FS:3 * FS]
    t_p1[1, ...] = y[:, 3 * FS:4 * FS]


def _table_shapes(n):
    return [jax.ShapeDtypeStruct((NC, n, FS), jnp.float32)] * 2


def _tf_pi_body(x_ref, w_ref, b_ref, t0, t1, s_ref):
    y = jnp.dot(x_ref[...], w_ref[...], preferred_element_type=jnp.float32)
    _emit_table(t0, t1, y[:, :HID])
    s_ref[...] = y[:, HID:2 * HID] + b_ref[...]


def _transform_pi(x, wcat, b):
    n = x.shape[0]
    return pl.pallas_call(
        _tf_pi_body,
        grid=(n // BROW,),
        in_specs=[_row_spec(DIM), _full_spec(DIM, 2 * HID), _full_spec(1, HID)],
        out_specs=[_AGG_SPEC] * 2 + [_row_spec(HID)],
        out_shape=_table_shapes(n) + [jax.ShapeDtypeStruct((n, HID), jnp.float32)],
    )(x, wcat, b)


def _tf_au_body(x_ref, w_ref, b_ref, a0, a1, w0, w1, s_ref):
    y = jnp.dot(x_ref[...], w_ref[...], preferred_element_type=jnp.float32)
    _emit_table(a0, a1, y[:, :HID])
    _emit_table(w0, w1, y[:, HID:2 * HID])
    s_ref[...] = y[:, 2 * HID:3 * HID] + b_ref[...]


def _transform_au(x, wcat, b):
    n = x.shape[0]
    return pl.pallas_call(
        _tf_au_body,
        grid=(n // BROW,),
        in_specs=[_row_spec(DIM), _full_spec(DIM, 3 * HID), _full_spec(1, HID)],
        out_specs=[_AGG_SPEC] * 4 + [_row_spec(HID)],
        out_shape=_table_shapes(n) * 2 + [jax.ShapeDtypeStruct((n, HID), jnp.float32)],
    )(x, wcat, b)


def _tf_pub_body(x_ref, w_ref, b_ref, cnt_ref, r0, r1, c0, c1, s_ref):
    y = jnp.dot(x_ref[...], w_ref[...], preferred_element_type=jnp.float32)
    deg = cnt_ref[0, :, 0:1] + cnt_ref[1, :, 0:1] + 1.0
    dinv = lax.rsqrt(deg)
    xw = y[:, HID:2 * HID]
    _emit_table(r0, r1, y[:, :HID])
    _emit_table(c0, c1, xw * dinv)
    s_ref[...] = y[:, 2 * HID:3 * HID] + xw * (dinv * dinv) + b_ref[...]


def _transform_pub(x, wcat, b, cnt):
    n = x.shape[0]
    return pl.pallas_call(
        _tf_pub_body,
        grid=(n // BROW,),
        in_specs=[_row_spec(DIM), _full_spec(DIM, 3 * HID), _full_spec(1, HID), _CNT_SPEC],
        out_specs=[_AGG_SPEC] * 4 + [_row_spec(HID)],
        out_shape=_table_shapes(n) * 2 + [jax.ShapeDtypeStruct((n, HID), jnp.float32)],
    )(x, wcat, b, cnt)


def _rc(cnt_ref):
    return 1.0 / jnp.maximum(cnt_ref[0, :, 0:1] + cnt_ref[1, :, 0:1], 1.0)


def _slices4(a_p0, a_p1):
    return (a_p0[0, ...], a_p0[1, ...], a_p1[0, ...], a_p1[1, ...])


def _comb1_body(a0, a1, cnt_ref, s_ref, o_ref):
    rc = _rc(cnt_ref)
    avs = _slices4(a0, a1)
    for i in range(4):
        o_ref[:, i * FS:(i + 1) * FS] = jnp.maximum(
            avs[i] * rc + s_ref[:, i * FS:(i + 1) * FS], 0.0)


def _combine_pi(aggs, cnt, s):
    n = s.shape[0]
    return pl.pallas_call(
        _comb1_body,
        grid=(n // BROW,),
        in_specs=[_AGG_SPEC] * 2 + [_CNT_SPEC, _row_spec(HID)],
        out_specs=_row_spec(HID),
        out_shape=jax.ShapeDtypeStruct((n, HID), jnp.float32),
    )(*aggs, cnt, s)


def _comb2_body(a0, a1, b0, b1, ca_ref, cb_ref, s_ref, o_ref):
    rca = _rc(ca_ref)
    rcb = _rc(cb_ref)
    avs = _slices4(a0, a1)
    bvs = _slices4(b0, b1)
    for i in range(4):
        o_ref[:, i * FS:(i + 1) * FS] = jnp.maximum(
            avs[i] * rca + bvs[i] * rcb + s_ref[:, i * FS:(i + 1) * FS], 0.0)


def _combine_au(aggs_pa, cpa, aggs_ra, cra, s):
    n = s.shape[0]
    return pl.pallas_call(
        _comb2_body,
        grid=(n // BROW,),
        in_specs=[_AGG_SPEC] * 4 + [_CNT_SPEC, _CNT_SPEC, _row_spec(HID)],
        out_specs=_row_spec(HID),
        out_shape=jax.ShapeDtypeStruct((n, HID), jnp.float32),
    )(*aggs_pa, *aggs_ra, cpa, cra, s)


def _combpub_body(a0, a1, g0, g1, caw_ref, ccc_ref, s_ref, o_ref):
    rc = _rc(caw_ref)
    dinv = lax.rsqrt(ccc_ref[0, :, 0:1] + ccc_ref[1, :, 0:1] + 1.0)
    avs = _slices4(a0, a1)
    gvs = _slices4(g0, g1)
    for i in range(4):
        o_ref[:, i * FS:(i + 1) * FS] = jnp.maximum(
            avs[i] * rc + gvs[i] * dinv + s_ref[:, i * FS:(i + 1) * FS], 0.0)


def _combine_pub(aggs_aw, caw, aggs_cc, ccc, s):
    n = s.shape[0]
    return pl.pallas_call(
        _combpub_body,
        grid=(n // BROW,),
        in_specs=[_AGG_SPEC] * 4 + [_CNT_SPEC, _CNT_SPEC, _row_spec(HID)],
        out_specs=_row_spec(HID),
        out_shape=jax.ShapeDtypeStruct((n, HID), jnp.float32),
    )(*aggs_aw, *aggs_cc, caw, ccc, s)


def _head_body(x_ref, h_ref, w1_ref, w2_ref, b_ref, o_ref):
    logits = (jnp.dot(x_ref[...], w1_ref[...], preferred_element_type=jnp.float32)
              + jnp.dot(h_ref[...], w2_ref[...], preferred_element_type=jnp.float32)
              + b_ref[...])
    m = jnp.max(logits, axis=-1, keepdims=True)
    e = jnp.exp(logits - m)
    o_ref[...] = e / jnp.sum(e, axis=-1, keepdims=True)


def _head(x, h, w1, w2, b):
    n = x.shape[0]
    return pl.pallas_call(
        _head_body,
        grid=(n // BROW,),
        in_specs=[_row_spec(DIM), _row_spec(HID), _full_spec(DIM, NOUT),
                  _full_spec(HID, NOUT), _full_spec(1, NOUT)],
        out_specs=_row_spec(NOUT),
        out_shape=jax.ShapeDtypeStruct((n, NOUT), jnp.float32),
    )(x, h, w1, w2, b)


def kernel(x_pi, x_author, x_pub, ei_pa, ei_ap, ei_aw, ei_ra, ei_cc, params):
    f32 = jnp.float32
    eis = {"pa": ei_pa, "ap": ei_ap, "aw": ei_aw, "ra": ei_ra, "cc": ei_cc}
    srcs2, dsts = {}, {}
    for key, e, ep, nd, _, ns_ in _RELS:
        ei = eis[key]
        pad = ep - e + 2 * KCH
        src = jnp.concatenate([ei[0].astype(jnp.int32), jnp.zeros((pad,), jnp.int32)])
        srcs2[key] = jnp.concatenate([src, src + ns_])
        dsts[key] = jnp.concatenate([ei[1].astype(jnp.int32), jnp.full((pad,), nd, jnp.int32)])
    ones8 = jnp.ones((KCNT, 8), f32)
    zeros8 = jnp.zeros((ZROWS, 8), f32)
    zeros32 = jnp.zeros((ZROWS, FS), f32)

    counts = _counts_call(dsts, ones8, zeros8)

    xs = {"pi": x_pi, "au": x_author, "pub": x_pub}
    for l in range(NLAYER):
        sfx = "%d" % l
        wcat_pi = jnp.concatenate([params["Wl_pa" + sfx], params["Wr_ap" + sfx]], axis=1)
        b_pi = params["bl_ap" + sfx].reshape(1, HID)
        tpi = _transform_pi(xs["pi"], wcat_pi, b_pi)
        t_pa, s_pi = tpi[:2], tpi[2]

        wcat_au = jnp.concatenate(
            [params["Wl_ap" + sfx], params["Wl_aw" + sfx],
             params["Wr_pa" + sfx] + params["Wr_ra" + sfx]], axis=1)
        b_au = (params["bl_pa" + sfx] + params["bl_ra" + sfx]).reshape(1, HID)
        tau = _transform_au(xs["au"], wcat_au, b_au)
        t_ap, t_aw, s_au = tau[:2], tau[2:4], tau[4]

        wcat_pub = jnp.concatenate(
            [params["Wl_ra" + sfx], params["Wg" + sfx], params["Wr_aw" + sfx]], axis=1)
        b_pub = (params["bl_aw" + sfx] + params["bg" + sfx]).reshape(1, HID)
        tpub = _transform_pub(xs["pub"], wcat_pub, b_pub, counts["cc"])
        t_ra, t_cc, s_pub = tpub[:2], tpub[2:4], tpub[4]

        def _flat(t):
            return tuple(a.reshape(NC * a.shape[1], FS) for a in t)

        tables = {"ap": _flat(t_ap), "pa": _flat(t_pa), "ra": _flat(t_ra),
                  "aw": _flat(t_aw), "cc": _flat(t_cc)}
        aggs = _segsum_call(tables, srcs2, dsts, zeros32)

        xs = {
            "pi": _combine_pi(aggs["ap"], counts["ap"], s_pi),
            "au": _combine_au(aggs["pa"], counts["pa"], aggs["ra"], counts["ra"], s_au),
            "pub": _combine_pub(aggs["aw"], counts["aw"], aggs["cc"], counts["cc"], s_pub),
        }

    probs = _head(x_pi, xs["pi"], params["Wf"][:DIM], params["Wf"][DIM:],
                  params["bf"].reshape(1, NOUT))
    return probs, xs["pi"], xs["au"], xs["pub"]


# speedup vs baseline: 1.0858x; 1.0858x over previous
import functools

import jax
import jax.numpy as jnp
from jax import lax
from jax.experimental import pallas as pl
from jax.experimental.pallas import tpu as pltpu
from jax.experimental.pallas import tpu_sc as plsc

NPI, NAU, NPUB = 10000, 20000, 50000
DIM, HID, NOUT, NLAYER = 128, 128, 4, 2
NC, NS = 2, 16
FS = 32
KCH = 384
KCNT = 192
BROW = 2000
ZROWS = 3200

NPAD_PI, NPAD_AU, NPAD_PUB = 10112, 20096, 50048

_RELS = (
    ("ap", 40000, 43008, NPI, NPAD_PI, NAU),
    ("pa", 40000, 43008, NAU, NPAD_AU, NPI),
    ("ra", 160000, 165888, NAU, NPAD_AU, NPUB),
    ("aw", 160000, 165888, NPUB, NPAD_PUB, NAU),
    ("cc", 200000, 202752, NPUB, NPAD_PUB, NPUB),
)
_RKEYS = tuple(r[0] for r in _RELS)

_SC_PARAMS = pltpu.CompilerParams(use_tc_tiling_on_sc=False)


@functools.cache
def _mesh():
    return plsc.VectorSubcoreMesh(core_axis_name="c", subcore_axis_name="s")


def _counts_body(*refs):
    it = iter(refs)
    dsts = {k: next(it) for k in _RKEYS}
    ones_hbm = next(it)
    zeros_hbm = next(it)
    outs = {k: next(it) for k in _RKEYS}
    didx, ones_v, cacc, sem = list(it)

    c = lax.axis_index("c")
    s = lax.axis_index("s")
    pltpu.sync_copy(ones_hbm, ones_v)

    for key, e, ep, nd, npad, ns_ in _RELS:
        steps = ep // (NC * NS * KCNT)
        eps = steps * KCNT
        base = (c * NS + s) * eps
        pltpu.sync_copy(dsts[key].at[pl.ds(base, eps)], didx.at[pl.ds(0, eps)])
        rps = npad // NS
        pltpu.sync_copy(zeros_hbm.at[pl.ds(0, rps)], cacc.at[pl.ds(s * rps, rps)])
        plsc.subcore_barrier()

        def step(j, carry):
            pltpu.sync_copy(ones_v, cacc.at[didx.at[pl.ds(j * KCNT, KCNT)]], add=True)
            return carry

        lax.fori_loop(0, steps, step, 0)
        plsc.subcore_barrier()
        pltpu.sync_copy(cacc.at[pl.ds(s * rps, rps)],
                        outs[key].at[c, pl.ds(s * rps, rps)])
        plsc.subcore_barrier()


def _counts_call(dsts, ones8, zeros8):
    out_types = tuple(jax.ShapeDtypeStruct((NC, npad, 8), jnp.float32)
                      for _, _, _, _, npad, _ in _RELS)
    max_eps_c = max(ep // (NC * NS) for _, _, ep, _, _, _ in _RELS)
    kern = functools.partial(
        pl.kernel,
        out_type=out_types,
        mesh=_mesh(),
        scratch_types=[
            pltpu.VMEM((max_eps_c,), jnp.int32),
            pltpu.VMEM((KCNT, 8), jnp.float32),
            pltpu.VMEM_SHARED((NPAD_PUB, 8), jnp.float32),
            pltpu.SemaphoreType.DMA,
        ],
        compiler_params=_SC_PARAMS,
    )(_counts_body)
    outs = kern(*[dsts[k] for k in _RKEYS], ones8, zeros8)
    return dict(zip(_RKEYS, outs))


def _make_segsum_body(rels):
    def _segsum_body(*refs):
        it = iter(refs)
        tables = {r[0]: (next(it), next(it)) for r in rels}
        edges = {r[0]: next(it) for r in rels}
        zeros_hbm = next(it)
        outs = {r[0]: (next(it), next(it)) for r in rels}
        eidxA, rowsA, eidxB, rowsB, acc, semA, semB = list(it)
        sA, dA = eidxA.at[pl.ds(0, KCH)], eidxA.at[pl.ds(KCH, KCH)]
        sB, dB = eidxB.at[pl.ds(0, KCH)], eidxB.at[pl.ds(KCH, KCH)]

        c = lax.axis_index("c")
        s = lax.axis_index("s")

        for key, e, ep, nd, npad, ns_ in rels:
            steps = ep // (NS * KCH)
            e2 = edges[key]
            rps = npad // NS
            nchunk = (ep + 2 * KCH) // KCH

            def fetch_idx(j, buf, _e2=e2, _steps=steps, _nc=nchunk):
                base = (c * _nc + s * _steps + j) * (2 * KCH)
                pltpu.sync_copy(_e2.at[pl.ds(base, 2 * KCH)], buf)

            for p in range(2):
                tbl = tables[key][p]
                out = outs[key][p]
                pltpu.sync_copy(zeros_hbm.at[pl.ds(0, rps)], acc.at[pl.ds(s * rps, rps)])
                plsc.subcore_barrier()

                fetch_idx(0, eidxA)
                pltpu.async_copy(tbl.at[sA], rowsA, semA)
                fetch_idx(1, eidxB)

                def step2(jj, carry, _tbl=tbl, _fi=fetch_idx):
                    pltpu.make_async_copy(_tbl.at[sA], rowsA, semA).wait()
                    pltpu.async_copy(_tbl.at[sB], rowsB, semB)
                    pltpu.sync_copy(rowsA, acc.at[dA], add=True)
                    _fi(2 * jj + 2, eidxA)
                    pltpu.async_copy(_tbl.at[sA], rowsA, semA)
                    pltpu.make_async_copy(_tbl.at[sB], rowsB, semB).wait()
                    pltpu.sync_copy(rowsB, acc.at[dB], add=True)
                    _fi(2 * jj + 3, eidxB)
                    return carry

                lax.fori_loop(0, steps // 2, step2, 0)
                pltpu.make_async_copy(tbl.at[sA], rowsA, semA).wait()
                if steps % 2 == 1:
                    pltpu.sync_copy(rowsA, acc.at[dA], add=True)
                plsc.subcore_barrier()
                pltpu.sync_copy(acc.at[pl.ds(s * rps, rps)],
                                out.at[c, pl.ds(s * rps, rps)])
                plsc.subcore_barrier()
    return _segsum_body


def _segsum_call(rels, tables, edges2, zeros32):
    out_types = []
    for _, _, _, _, npad, _ in rels:
        out_types += [jax.ShapeDtypeStruct((NC, npad, FS), jnp.float32)] * 2
    kern = functools.partial(
        pl.kernel,
        out_type=tuple(out_types),
        mesh=_mesh(),
        scratch_types=[
            pltpu.VMEM((2 * KCH,), jnp.int32),
            pltpu.VMEM((KCH, FS), jnp.float32),
            pltpu.VMEM((2 * KCH,), jnp.int32),
            pltpu.VMEM((KCH, FS), jnp.float32),
            pltpu.VMEM_SHARED((NPAD_PUB, FS), jnp.float32),
            pltpu.SemaphoreType.DMA,
            pltpu.SemaphoreType.DMA,
        ],
        compiler_params=_SC_PARAMS,
    )(_make_segsum_body(rels))
    args = []
    for r in rels:
        args += list(tables[r[0]])
    for r in rels:
        args.append(edges2[r[0]])
    args.append(zeros32)
    flat = kern(*args)
    return {r[0]: (flat[2 * i], flat[2 * i + 1]) for i, r in enumerate(rels)}


def _row_spec(w):
    return pl.BlockSpec((BROW, w), lambda i: (i, 0))


def _full_spec(h, w):
    return pl.BlockSpec((h, w), lambda i: (0, 0))


_CNT_SPEC = pl.BlockSpec((NC, BROW, 8), lambda i: (0, i, 0))
_AGG_SPEC = pl.BlockSpec((NC, BROW, FS), lambda i: (0, i, 0))


def _emit_table(t_p0, t_p1, y):
    t_p0[0, ...] = y[:, 0 * FS:1 * FS]
    t_p0[1, ...] = y[:, 1 * FS:2 * FS]
    t_p1[0, ...] = y[:, 2 * FS:3 * FS]
    t_p1[1, ...] = y[:, 3 * FS:4 * FS]


def _table_shapes(n):
    return [jax.ShapeDtypeStruct((NC, n, FS), jnp.float32)] * 2


def _tf_pi_body(x_ref, w_ref, b_ref, t0, t1, s_ref):
    y = jnp.dot(x_ref[...], w_ref[...], preferred_element_type=jnp.float32)
    _emit_table(t0, t1, y[:, :HID])
    s_ref[...] = y[:, HID:2 * HID] + b_ref[...]


def _transform_pi(x, wcat, b):
    n = x.shape[0]
    return pl.pallas_call(
        _tf_pi_body,
        grid=(n // BROW,),
        in_specs=[_row_spec(DIM), _full_spec(DIM, 2 * HID), _full_spec(1, HID)],
        out_specs=[_AGG_SPEC] * 2 + [_row_spec(HID)],
        out_shape=_table_shapes(n) + [jax.ShapeDtypeStruct((n, HID), jnp.float32)],
    )(x, wcat, b)


def _tf_au_body(x_ref, w_ref, b_ref, a0, a1, w0, w1, s_ref):
    y = jnp.dot(x_ref[...], w_ref[...], preferred_element_type=jnp.float32)
    _emit_table(a0, a1, y[:, :HID])
    _emit_table(w0, w1, y[:, HID:2 * HID])
    s_ref[...] = y[:, 2 * HID:3 * HID] + b_ref[...]


def _transform_au(x, wcat, b):
    n = x.shape[0]
    return pl.pallas_call(
        _tf_au_body,
        grid=(n // BROW,),
        in_specs=[_row_spec(DIM), _full_spec(DIM, 3 * HID), _full_spec(1, HID)],
        out_specs=[_AGG_SPEC] * 4 + [_row_spec(HID)],
        out_shape=_table_shapes(n) * 2 + [jax.ShapeDtypeStruct((n, HID), jnp.float32)],
    )(x, wcat, b)


def _tf_pub_body(x_ref, w_ref, b_ref, cnt_ref, r0, r1, c0, c1, s_ref):
    y = jnp.dot(x_ref[...], w_ref[...], preferred_element_type=jnp.float32)
    deg = cnt_ref[0, :, 0:1] + cnt_ref[1, :, 0:1] + 1.0
    dinv = lax.rsqrt(deg)
    xw = y[:, HID:2 * HID]
    _emit_table(r0, r1, y[:, :HID])
    _emit_table(c0, c1, xw * dinv)
    s_ref[...] = y[:, 2 * HID:3 * HID] + xw * (dinv * dinv) + b_ref[...]


def _transform_pub(x, wcat, b, cnt):
    n = x.shape[0]
    return pl.pallas_call(
        _tf_pub_body,
        grid=(n // BROW,),
        in_specs=[_row_spec(DIM), _full_spec(DIM, 3 * HID), _full_spec(1, HID), _CNT_SPEC],
        out_specs=[_AGG_SPEC] * 4 + [_row_spec(HID)],
        out_shape=_table_shapes(n) * 2 + [jax.ShapeDtypeStruct((n, HID), jnp.float32)],
    )(x, wcat, b, cnt)


def _rc(cnt_ref):
    return 1.0 / jnp.maximum(cnt_ref[0, :, 0:1] + cnt_ref[1, :, 0:1], 1.0)


def _slices4(a_p0, a_p1):
    return (a_p0[0, ...], a_p0[1, ...], a_p1[0, ...], a_p1[1, ...])


def _comb1_body(a0, a1, cnt_ref, s_ref, o_ref):
    rc = _rc(cnt_ref)
    avs = _slices4(a0, a1)
    for i in range(4):
        o_ref[:, i * FS:(i + 1) * FS] = jnp.maximum(
            avs[i] * rc + s_ref[:, i * FS:(i + 1) * FS], 0.0)


def _combine_pi(aggs, cnt, s):
    n = s.shape[0]
    return pl.pallas_call(
        _comb1_body,
        grid=(n // BROW,),
        in_specs=[_AGG_SPEC] * 2 + [_CNT_SPEC, _row_spec(HID)],
        out_specs=_row_spec(HID),
        out_shape=jax.ShapeDtypeStruct((n, HID), jnp.float32),
    )(*aggs, cnt, s)


def _comb2_body(a0, a1, b0, b1, ca_ref, cb_ref, s_ref, o_ref):
    rca = _rc(ca_ref)
    rcb = _rc(cb_ref)
    avs = _slices4(a0, a1)
    bvs = _slices4(b0, b1)
    for i in range(4):
        o_ref[:, i * FS:(i + 1) * FS] = jnp.maximum(
            avs[i] * rca + bvs[i] * rcb + s_ref[:, i * FS:(i + 1) * FS], 0.0)


def _combine_au(aggs_pa, cpa, aggs_ra, cra, s):
    n = s.shape[0]
    return pl.pallas_call(
        _comb2_body,
        grid=(n // BROW,),
        in_specs=[_AGG_SPEC] * 4 + [_CNT_SPEC, _CNT_SPEC, _row_spec(HID)],
        out_specs=_row_spec(HID),
        out_shape=jax.ShapeDtypeStruct((n, HID), jnp.float32),
    )(*aggs_pa, *aggs_ra, cpa, cra, s)


def _combpub_body(a0, a1, g0, g1, caw_ref, ccc_ref, s_ref, o_ref):
    rc = _rc(caw_ref)
    dinv = lax.rsqrt(ccc_ref[0, :, 0:1] + ccc_ref[1, :, 0:1] + 1.0)
    avs = _slices4(a0, a1)
    gvs = _slices4(g0, g1)
    for i in range(4):
        o_ref[:, i * FS:(i + 1) * FS] = jnp.maximum(
            avs[i] * rc + gvs[i] * dinv + s_ref[:, i * FS:(i + 1) * FS], 0.0)


def _combine_pub(aggs_aw, caw, aggs_cc, ccc, s):
    n = s.shape[0]
    return pl.pallas_call(
        _combpub_body,
        grid=(n // BROW,),
        in_specs=[_AGG_SPEC] * 4 + [_CNT_SPEC, _CNT_SPEC, _row_spec(HID)],
        out_specs=_row_spec(HID),
        out_shape=jax.ShapeDtypeStruct((n, HID), jnp.float32),
    )(*aggs_aw, *aggs_cc, caw, ccc, s)


def _head_body(x_ref, h_ref, w1_ref, w2_ref, b_ref, o_ref):
    logits = (jnp.dot(x_ref[...], w1_ref[...], preferred_element_type=jnp.float32)
              + jnp.dot(h_ref[...], w2_ref[...], preferred_element_type=jnp.float32)
              + b_ref[...])
    m = jnp.max(logits, axis=-1, keepdims=True)
    e = jnp.exp(logits - m)
    o_ref[...] = e / jnp.sum(e, axis=-1, keepdims=True)


def _head(x, h, w1, w2, b):
    n = x.shape[0]
    return pl.pallas_call(
        _head_body,
        grid=(n // BROW,),
        in_specs=[_row_spec(DIM), _row_spec(HID), _full_spec(DIM, NOUT),
                  _full_spec(HID, NOUT), _full_spec(1, NOUT)],
        out_specs=_row_spec(NOUT),
        out_shape=jax.ShapeDtypeStruct((n, NOUT), jnp.float32),
    )(x, h, w1, w2, b)


def kernel(x_pi, x_author, x_pub, ei_pa, ei_ap, ei_aw, ei_ra, ei_cc, params):
    f32 = jnp.float32
    eis = {"pa": ei_pa, "ap": ei_ap, "aw": ei_aw, "ra": ei_ra, "cc": ei_cc}
    edges2, dsts = {}, {}
    for key, e, ep, nd, _, ns_ in _RELS:
        ei = eis[key]
        pad = ep - e + 2 * KCH
        srcp = jnp.concatenate([ei[0].astype(jnp.int32), jnp.zeros((pad,), jnp.int32)])
        dstp = jnp.concatenate([ei[1].astype(jnp.int32), jnp.full((pad,), nd, jnp.int32)])
        dsts[key] = dstp
        nchunk = (ep + 2 * KCH) // KCH
        per_core = []
        for cc in range(NC):
            sc_ = (srcp + cc * ns_).reshape(nchunk, 1, KCH)
            dc_ = dstp.reshape(nchunk, 1, KCH)
            per_core.append(jnp.concatenate([sc_, dc_], axis=1))
        edges2[key] = jnp.stack(per_core).reshape(-1)
    ones8 = jnp.ones((KCNT, 8), f32)
    zeros8 = jnp.zeros((ZROWS, 8), f32)
    zeros32 = jnp.zeros((ZROWS, FS), f32)

    counts = _counts_call(dsts, ones8, zeros8)

    xs = {"pi": x_pi, "au": x_author, "pub": x_pub}
    for l in range(NLAYER):
        sfx = "%d" % l
        wcat_pi = jnp.concatenate([params["Wl_pa" + sfx], params["Wr_ap" + sfx]], axis=1)
        b_pi = params["bl_ap" + sfx].reshape(1, HID)
        tpi = _transform_pi(xs["pi"], wcat_pi, b_pi)
        t_pa, s_pi = tpi[:2], tpi[2]

        wcat_au = jnp.concatenate(
            [params["Wl_ap" + sfx], params["Wl_aw" + sfx],
             params["Wr_pa" + sfx] + params["Wr_ra" + sfx]], axis=1)
        b_au = (params["bl_pa" + sfx] + params["bl_ra" + sfx]).reshape(1, HID)
        tau = _transform_au(xs["au"], wcat_au, b_au)
        t_ap, t_aw, s_au = tau[:2], tau[2:4], tau[4]

        wcat_pub = jnp.concatenate(
            [params["Wl_ra" + sfx], params["Wg" + sfx], params["Wr_aw" + sfx]], axis=1)
        b_pub = (params["bl_aw" + sfx] + params["bg" + sfx]).reshape(1, HID)
        tpub = _transform_pub(xs["pub"], wcat_pub, b_pub, counts["cc"])
        t_ra, t_cc, s_pub = tpub[:2], tpub[2:4], tpub[4]

        def _flat(t):
            return tuple(a.reshape(NC * a.shape[1], FS) for a in t)

        tables = {"ap": _flat(t_ap), "pa": _flat(t_pa), "ra": _flat(t_ra),
                  "aw": _flat(t_aw), "cc": _flat(t_cc)}
        aggs_a = _segsum_call(_RELS[:3], tables, edges2, zeros32)
        aggs_b = _segsum_call(_RELS[3:], tables, edges2, zeros32)

        xs = {
            "pi": _combine_pi(aggs_a["ap"], counts["ap"], s_pi),
            "au": _combine_au(aggs_a["pa"], counts["pa"], aggs_a["ra"], counts["ra"], s_au),
            "pub": _combine_pub(aggs_b["aw"], counts["aw"], aggs_b["cc"], counts["cc"], s_pub),
        }

    probs = _head(x_pi, xs["pi"], params["Wf"][:DIM], params["Wf"][DIM:],
                  params["bf"].reshape(1, NOUT))
    return probs, xs["pi"], xs["au"], xs["pub"]
